# Initial kernel scaffold; baseline (speedup 1.0000x reference)
import jax
import jax.numpy as jnp
from jax.experimental import pallas as pl


def kernel(adv_patch, boxes_batch, base):
    raise NotImplementedError("write your pallas kernel here")


# speedup vs baseline: 3.9198x; 3.9198x over previous
import jax
import jax.numpy as jnp
import numpy as np
from jax import lax
from jax.experimental import pallas as pl

_IMG = 512
_PH, _PW = 64, 64
_BATCH = 32


def _nn_idx_table(in_size):
    t = np.zeros((_IMG + 1, _IMG), dtype=np.int32)
    for s in range(1, _IMG + 1):
        t[s, :s] = np.minimum(
            (np.arange(s) * (in_size / s)).astype(np.int32), in_size - 1)
    return jnp.asarray(t)


_ROW_TABLE = _nn_idx_table(_PH)
_COL_TABLE = _nn_idx_table(_PW)


def _placement(boxes_batch):
    box = jnp.clip(boxes_batch[:, 0], 0, _IMG).astype(jnp.int32)
    midx = (box[:, 3] + box[:, 1]) // 2
    midy = (box[:, 2] + box[:, 0]) // 2
    y2x = _PW / _PH
    xs_a = jnp.floor((box[:, 3] - box[:, 1]).astype(jnp.float32)).astype(jnp.int32)
    xs_b = jnp.floor((box[:, 2] - box[:, 0]).astype(jnp.float32) / y2x).astype(jnp.int32)
    xsize = jnp.maximum(jnp.minimum(xs_a, xs_b), 1)
    ysize = jnp.maximum(jnp.floor(y2x * xsize.astype(jnp.float32)).astype(jnp.int32), 1)
    x1 = jnp.clip(midx - xsize // 2, 0, _IMG - xsize)
    y1 = jnp.clip(midy - ysize // 2, 0, _IMG - ysize)
    px = jnp.arange(_IMG, dtype=jnp.int32)[None, :]
    i = px - x1[:, None]
    j = px - y1[:, None]
    xi = _ROW_TABLE[xsize[:, None], jnp.clip(i, 0, _IMG - 1)]
    yi = _COL_TABLE[ysize[:, None], jnp.clip(j, 0, _IMG - 1)]
    rowsel = jnp.where((i >= 0) & (i < xsize[:, None]), xi, -1)
    colsel = jnp.where((j >= 0) & (j < ysize[:, None]), yi, -1)
    return rowsel, colsel


def _body(patch_ref, base_ref, rs_ref, cs_ref, out_ref):
    xi = rs_ref[0, 0, :]
    yi = cs_ref[0, 0, :]
    k = lax.broadcasted_iota(jnp.int32, (_IMG, _PH), 1)
    roh = (xi[:, None] == k).astype(jnp.float32)
    coh = (yi[:, None] == k).astype(jnp.float32)
    for c in range(3):
        p = patch_ref[c, :, :]
        t = lax.dot(roh, p, precision=lax.Precision.HIGHEST)
        g = lax.dot_general(t, coh, (((1,), (1,)), ((), ())),
                            precision=lax.Precision.HIGHEST)
        out_ref[0, c, :, :] = jnp.where(g != 0.0, g, base_ref[c, :, :])


def kernel(adv_patch, boxes_batch, base):
    rowsel, colsel = _placement(boxes_batch)
    rowsel = rowsel.reshape(_BATCH, 1, _IMG)
    colsel = colsel.reshape(_BATCH, 1, _IMG)
    return pl.pallas_call(
        _body,
        grid=(_BATCH,),
        in_specs=[
            pl.BlockSpec((3, _PH, _PW), lambda b: (0, 0, 0)),
            pl.BlockSpec((3, _IMG, _IMG), lambda b: (0, 0, 0)),
            pl.BlockSpec((1, 1, _IMG), lambda b: (b, 0, 0)),
            pl.BlockSpec((1, 1, _IMG), lambda b: (b, 0, 0)),
        ],
        out_specs=pl.BlockSpec((1, 3, _IMG, _IMG), lambda b: (b, 0, 0, 0)),
        out_shape=jax.ShapeDtypeStruct((_BATCH, 3, _IMG, _IMG), jnp.float32),
    )(adv_patch, base, rowsel, colsel)
